# baseline (speedup 1.0000x reference)
import functools

import jax
import jax.numpy as jnp
from jax import lax
from jax.experimental import pallas as pl
from jax.experimental.pallas import tpu as pltpu
from jax.experimental.pallas import tpu_sc as plsc

_B, _C, _F = 1024, 2, 4096
_ROW = _C * _F
_BB = 64
_SC_ROWS = 256
_SC_CORES = 1
_NBUF = 4
_CH_ROWS = 1
_CH = _CH_ROWS * _ROW
_GRP = 2 * 128


def _tc_body(p_ref, t_ref, o_ref):
    i = pl.program_id(0)
    ps = p_ref[:, 0, :]
    pc = p_ref[:, 1, :]
    ts = t_ref[:, 0, :]
    tc = t_ref[:, 1, :]
    m = tc != 0.0
    term = jnp.where(m, (ps - ts) ** 2 + (pc - tc) ** 2, 0.0)
    partial = jnp.sum(term)

    @pl.when(i == 0)
    def _():
        o_ref[0, 0] = 0.0

    o_ref[0, 0] += partial


def _tc_loss(pred_o, target_o, row0, row1=_B):
    assert row0 % _BB == 0 and (row1 - row0) % _BB == 0
    grid = (row1 - row0) // _BB
    g0 = row0 // _BB
    out = pl.pallas_call(
        _tc_body,
        grid=(grid,),
        in_specs=[
            pl.BlockSpec((_BB, _C, _F), lambda i: (i + g0, 0, 0)),
            pl.BlockSpec((_BB, _C, _F), lambda i: (i + g0, 0, 0)),
        ],
        out_specs=pl.BlockSpec(memory_space=pltpu.SMEM),
        out_shape=jax.ShapeDtypeStruct((1, 1), jnp.float32),
    )(pred_o, target_o)
    return out[0, 0]


def _sc_partials(p_flat, t_flat, sc_rows):
    info = plsc.get_sparse_core_info()
    nc, ns = _SC_CORES, info.num_subcores
    nw = nc * ns
    w_floats = (sc_rows // nw) * _ROW
    nch = w_floats // _CH
    mesh = plsc.VectorSubcoreMesh(
        core_axis_name="c", subcore_axis_name="s", num_cores=nc
    )

    @functools.partial(
        pl.kernel,
        mesh=mesh,
        compiler_params=pltpu.CompilerParams(skip_device_barrier=True),
        out_type=jax.ShapeDtypeStruct((nw, 16), jnp.float32),
        scratch_types=(
            [pltpu.VMEM((_CH,), jnp.float32) for _ in range(2 * _NBUF)]
            + [pltpu.VMEM((16,), jnp.float32)]
            + [pltpu.SemaphoreType.DMA for _ in range(2 * _NBUF)]
        ),
    )
    def k(p_hbm, t_hbm, out_hbm, *rest):
        pbufs = rest[0:_NBUF]
        tbufs = rest[_NBUF : 2 * _NBUF]
        accb = rest[2 * _NBUF]
        psems = rest[2 * _NBUF + 1 : 3 * _NBUF + 1]
        tsems = rest[3 * _NBUF + 1 : 4 * _NBUF + 1]
        wid = lax.axis_index("s") * nc + lax.axis_index("c")
        base = wid * w_floats

        def start(g, b):
            off = base + g * _CH
            pltpu.async_copy(p_hbm.at[pl.ds(off, _CH)], pbufs[b], psems[b])
            pltpu.async_copy(t_hbm.at[pl.ds(off, _CH)], tbufs[b], tsems[b])

        def wait(b):
            pltpu.make_async_copy(p_hbm.at[pl.ds(0, _CH)], pbufs[b], psems[b]).wait()
            pltpu.make_async_copy(t_hbm.at[pl.ds(0, _CH)], tbufs[b], tsems[b]).wait()

        def chunk_sum(b, acc):
            pb, tb = pbufs[b], tbufs[b]

            def inner(j, a):
                goff = j * _GRP
                for kk in range(8):
                    offs = goff + kk * 16
                    offc = offs + 128
                    ps = pb[pl.ds(offs, 16)]
                    ts = tb[pl.ds(offs, 16)]
                    pc = pb[pl.ds(offc, 16)]
                    tc = tb[pl.ds(offc, 16)]
                    es = ps - ts
                    ec = pc - tc
                    d2 = es * es + ec * ec
                    a = a + jnp.where(tc != 0.0, d2, 0.0)
                return a

            return lax.fori_loop(0, _CH // _GRP, inner, acc)

        nsteady = nch - _NBUF
        iters = nsteady // _NBUF
        rem = nsteady % _NBUF
        for b in range(min(_NBUF, nch)):
            start(b, b)

        def body(i, acc):
            for b in range(_NBUF):
                wait(b)
                acc = chunk_sum(b, acc)
                start(i * _NBUF + b + _NBUF, b)
            return acc

        acc = lax.fori_loop(0, iters, body, jnp.zeros((16,), jnp.float32))
        for j in range(rem):
            wait(j)
            acc = chunk_sum(j, acc)
            start(iters * _NBUF + j + _NBUF, j)
        for j in range(min(_NBUF, nch)):
            b = (rem + j) % _NBUF
            wait(b)
            acc = chunk_sum(b, acc)
        accb[...] = acc
        pltpu.sync_copy(accb, out_hbm.at[wid])

    return k(p_flat, t_flat)


def _physical_view(x):
    return x.reshape(_B, _C, _F // 128, 128).transpose(0, 2, 1, 3).reshape(-1)


def kernel(pred_o, target_o):
    mid = _SC_ROWS + 256
    tc_a = _tc_loss(pred_o, target_o, _SC_ROWS, mid)
    sc_out = _sc_partials(
        _physical_view(pred_o), _physical_view(target_o), _SC_ROWS
    )
    tc_b = _tc_loss(pred_o, target_o, mid)
    return jnp.sum(sc_out) + tc_a + tc_b


# speedup vs baseline: 1.0542x; 1.0542x over previous
import functools

import jax
import jax.numpy as jnp
from jax import lax
from jax.experimental import pallas as pl
from jax.experimental.pallas import tpu as pltpu
from jax.experimental.pallas import tpu_sc as plsc

_B, _C, _F = 1024, 2, 4096
_ROW = _C * _F
_BB = 64
_SC_ROWS = 256
_SC_CORES = 1
_NBUF = 4
_CH_ROWS = 1
_CH = _CH_ROWS * _ROW
_GRP = 2 * 128


def _tc_body(p_ref, t_ref, o_ref):
    i = pl.program_id(0)
    ps = p_ref[:, 0, :]
    pc = p_ref[:, 1, :]
    ts = t_ref[:, 0, :]
    tc = t_ref[:, 1, :]
    m = tc != 0.0
    term = jnp.where(m, (ps - ts) ** 2 + (pc - tc) ** 2, 0.0)
    partial = jnp.sum(term)

    @pl.when(i == 0)
    def _():
        o_ref[0, 0] = 0.0

    o_ref[0, 0] += partial


def _tc_loss(pred_o, target_o, row0, row1=_B):
    assert row0 % _BB == 0 and (row1 - row0) % _BB == 0
    grid = (row1 - row0) // _BB
    g0 = row0 // _BB
    out = pl.pallas_call(
        _tc_body,
        grid=(grid,),
        in_specs=[
            pl.BlockSpec((_BB, _C, _F), lambda i: (i + g0, 0, 0)),
            pl.BlockSpec((_BB, _C, _F), lambda i: (i + g0, 0, 0)),
        ],
        out_specs=pl.BlockSpec(memory_space=pltpu.SMEM),
        out_shape=jax.ShapeDtypeStruct((1, 1), jnp.float32),
    )(pred_o, target_o)
    return out[0, 0]


def _sc_partials(p_flat, t_flat, sc_rows):
    info = plsc.get_sparse_core_info()
    nc, ns = _SC_CORES, info.num_subcores
    nw = nc * ns
    w_floats = (sc_rows // nw) * _ROW
    nch = w_floats // _CH
    mesh = plsc.VectorSubcoreMesh(
        core_axis_name="c", subcore_axis_name="s", num_cores=nc
    )

    @functools.partial(
        pl.kernel,
        mesh=mesh,
        compiler_params=pltpu.CompilerParams(skip_device_barrier=True),
        out_type=jax.ShapeDtypeStruct((nw, 16), jnp.float32),
        scratch_types=(
            [pltpu.VMEM((_CH,), jnp.float32) for _ in range(2 * _NBUF)]
            + [pltpu.VMEM((16,), jnp.float32)]
            + [pltpu.SemaphoreType.DMA for _ in range(2 * _NBUF)]
        ),
    )
    def k(p_hbm, t_hbm, out_hbm, *rest):
        pbufs = rest[0:_NBUF]
        tbufs = rest[_NBUF : 2 * _NBUF]
        accb = rest[2 * _NBUF]
        psems = rest[2 * _NBUF + 1 : 3 * _NBUF + 1]
        tsems = rest[3 * _NBUF + 1 : 4 * _NBUF + 1]
        wid = lax.axis_index("s") * nc + lax.axis_index("c")
        base = wid * w_floats

        def start(g, b):
            off = base + g * _CH
            pltpu.async_copy(p_hbm.at[pl.ds(off, _CH)], pbufs[b], psems[b])
            pltpu.async_copy(t_hbm.at[pl.ds(off, _CH)], tbufs[b], tsems[b])

        def wait(b):
            pltpu.make_async_copy(p_hbm.at[pl.ds(0, _CH)], pbufs[b], psems[b]).wait()
            pltpu.make_async_copy(t_hbm.at[pl.ds(0, _CH)], tbufs[b], tsems[b]).wait()

        def chunk_sum(b, acc):
            pb, tb = pbufs[b], tbufs[b]

            def inner(j, a):
                goff = j * _GRP
                for kk in range(8):
                    offs = goff + kk * 16
                    offc = offs + 128
                    ps = pb[pl.ds(offs, 16)]
                    ts = tb[pl.ds(offs, 16)]
                    pc = pb[pl.ds(offc, 16)]
                    tc = tb[pl.ds(offc, 16)]
                    es = ps - ts
                    ec = pc - tc
                    d2 = es * es + ec * ec
                    a = a + jnp.where(tc != 0.0, d2, 0.0)
                return a

            return lax.fori_loop(0, _CH // _GRP, inner, acc)

        nsteady = nch - _NBUF
        iters = nsteady // _NBUF
        rem = nsteady % _NBUF
        for b in range(min(_NBUF, nch)):
            start(b, b)

        def body(i, acc):
            for b in range(_NBUF):
                wait(b)
                acc = chunk_sum(b, acc)
                start(i * _NBUF + b + _NBUF, b)
            return acc

        acc = lax.fori_loop(0, iters, body, jnp.zeros((16,), jnp.float32))
        for j in range(rem):
            wait(j)
            acc = chunk_sum(j, acc)
            start(iters * _NBUF + j + _NBUF, j)
        for j in range(min(_NBUF, nch)):
            b = (rem + j) % _NBUF
            wait(b)
            acc = chunk_sum(b, acc)
        accb[...] = acc
        pltpu.sync_copy(accb, out_hbm.at[wid])

    return k(p_flat, t_flat)


def _physical_view(x):
    return x.reshape(_B, _C, _F // 128, 128).transpose(0, 2, 1, 3).reshape(-1)


def kernel(pred_o, target_o):
    sc_out = _sc_partials(
        _physical_view(pred_o), _physical_view(target_o), _SC_ROWS
    )
    tc_part = _tc_loss(pred_o, target_o, _SC_ROWS)
    return jnp.sum(sc_out) + tc_part


# speedup vs baseline: 1.0759x; 1.0206x over previous
import functools

import jax
import jax.numpy as jnp
from jax import lax
from jax.experimental import pallas as pl
from jax.experimental.pallas import tpu as pltpu
from jax.experimental.pallas import tpu_sc as plsc

_B, _C, _F = 1024, 2, 4096
_ROW = _C * _F
_BB = 64
_SC_ROWS = 320
_SC_CORES = 1
_NBUF = 4
_CH_ROWS = 1
_CH = _CH_ROWS * _ROW
_GRP = 2 * 128


def _tc_body(p_ref, t_ref, o_ref):
    i = pl.program_id(0)
    ps = p_ref[:, 0, :]
    pc = p_ref[:, 1, :]
    ts = t_ref[:, 0, :]
    tc = t_ref[:, 1, :]
    m = tc != 0.0
    term = jnp.where(m, (ps - ts) ** 2 + (pc - tc) ** 2, 0.0)
    partial = jnp.sum(term)

    @pl.when(i == 0)
    def _():
        o_ref[0, 0] = 0.0

    o_ref[0, 0] += partial


def _tc_loss(pred_o, target_o, row0, row1=_B):
    assert row0 % _BB == 0 and (row1 - row0) % _BB == 0
    grid = (row1 - row0) // _BB
    g0 = row0 // _BB
    out = pl.pallas_call(
        _tc_body,
        grid=(grid,),
        in_specs=[
            pl.BlockSpec((_BB, _C, _F), lambda i: (i + g0, 0, 0)),
            pl.BlockSpec((_BB, _C, _F), lambda i: (i + g0, 0, 0)),
        ],
        out_specs=pl.BlockSpec(memory_space=pltpu.SMEM),
        out_shape=jax.ShapeDtypeStruct((1, 1), jnp.float32),
    )(pred_o, target_o)
    return out[0, 0]


def _sc_partials(p_flat, t_flat, sc_rows):
    info = plsc.get_sparse_core_info()
    nc, ns = _SC_CORES, info.num_subcores
    nw = nc * ns
    w_floats = (sc_rows // nw) * _ROW
    nch = w_floats // _CH
    mesh = plsc.VectorSubcoreMesh(
        core_axis_name="c", subcore_axis_name="s", num_cores=nc
    )

    @functools.partial(
        pl.kernel,
        mesh=mesh,
        compiler_params=pltpu.CompilerParams(skip_device_barrier=True),
        out_type=jax.ShapeDtypeStruct((nw, 16), jnp.float32),
        scratch_types=(
            [pltpu.VMEM((_CH,), jnp.float32) for _ in range(2 * _NBUF)]
            + [pltpu.VMEM((16,), jnp.float32)]
            + [pltpu.SemaphoreType.DMA for _ in range(2 * _NBUF)]
        ),
    )
    def k(p_hbm, t_hbm, out_hbm, *rest):
        pbufs = rest[0:_NBUF]
        tbufs = rest[_NBUF : 2 * _NBUF]
        accb = rest[2 * _NBUF]
        psems = rest[2 * _NBUF + 1 : 3 * _NBUF + 1]
        tsems = rest[3 * _NBUF + 1 : 4 * _NBUF + 1]
        wid = lax.axis_index("s") * nc + lax.axis_index("c")
        base = wid * w_floats

        def start(g, b):
            off = base + g * _CH
            pltpu.async_copy(p_hbm.at[pl.ds(off, _CH)], pbufs[b], psems[b])
            pltpu.async_copy(t_hbm.at[pl.ds(off, _CH)], tbufs[b], tsems[b])

        def wait(b):
            pltpu.make_async_copy(p_hbm.at[pl.ds(0, _CH)], pbufs[b], psems[b]).wait()
            pltpu.make_async_copy(t_hbm.at[pl.ds(0, _CH)], tbufs[b], tsems[b]).wait()

        def chunk_sum(b, acc):
            pb, tb = pbufs[b], tbufs[b]

            def inner(j, a):
                goff = j * _GRP
                for kk in range(8):
                    offs = goff + kk * 16
                    offc = offs + 128
                    ps = pb[pl.ds(offs, 16)]
                    ts = tb[pl.ds(offs, 16)]
                    pc = pb[pl.ds(offc, 16)]
                    tc = tb[pl.ds(offc, 16)]
                    es = ps - ts
                    ec = pc - tc
                    d2 = es * es + ec * ec
                    a = a + jnp.where(tc != 0.0, d2, 0.0)
                return a

            return lax.fori_loop(0, _CH // _GRP, inner, acc)

        nsteady = nch - _NBUF
        iters = nsteady // _NBUF
        rem = nsteady % _NBUF
        for b in range(min(_NBUF, nch)):
            start(b, b)

        def body(i, acc):
            for b in range(_NBUF):
                wait(b)
                acc = chunk_sum(b, acc)
                start(i * _NBUF + b + _NBUF, b)
            return acc

        acc = lax.fori_loop(0, iters, body, jnp.zeros((16,), jnp.float32))
        for j in range(rem):
            wait(j)
            acc = chunk_sum(j, acc)
            start(iters * _NBUF + j + _NBUF, j)
        for j in range(min(_NBUF, nch)):
            b = (rem + j) % _NBUF
            wait(b)
            acc = chunk_sum(b, acc)
        accb[...] = acc
        pltpu.sync_copy(accb, out_hbm.at[wid])

    return k(p_flat, t_flat)


def _physical_view(x):
    return x.reshape(_B, _C, _F // 128, 128).transpose(0, 2, 1, 3).reshape(-1)


def kernel(pred_o, target_o):
    sc_out = _sc_partials(
        _physical_view(pred_o), _physical_view(target_o), _SC_ROWS
    )
    tc_part = _tc_loss(pred_o, target_o, _SC_ROWS)
    return jnp.sum(sc_out) + tc_part
